# Initial kernel scaffold; baseline (speedup 1.0000x reference)
import functools

import jax
import jax.numpy as jnp
from jax import lax
from jax.experimental import pallas as pl
from jax.experimental.pallas import tpu as pltpu
from jax.experimental.pallas import tpu_sc as plsc

MAXP = 100000
D = 64
B_TOTAL = 4096 * 200
NC, NS, L = 2, 16, 16
NW = NC * NS
B_PER_W = B_TOTAL // NW
IDXW = 128
NSTREAM = 8
CHUNK = NSTREAM * IDXW
NCHUNK = B_PER_W // CHUNK
IROWS_PER_W = B_PER_W // IDXW


def _posenc_body(t2, table, out, idx_v, rows_v, isem, gsem, osem):
    wid = lax.axis_index("s") * NC + lax.axis_index("c")
    base = wid * B_PER_W
    irow0 = wid * IROWS_PER_W

    def chunk_body(ci, carry):
        pltpu.async_copy(
            t2.at[pl.ds(irow0 + ci * NSTREAM, NSTREAM)], idx_v, isem
        ).wait()
        for j in range(NSTREAM):
            for i in range(IDXW // L):
                v = idx_v[j, pl.ds(i * L, L)]
                idx_v[j, pl.ds(i * L, L)] = jnp.where(v == 0, MAXP - 1, v - 1)
        handles = [
            pltpu.async_copy(
                table.at[idx_v.at[j]],
                rows_v.at[pl.ds(j * IDXW, IDXW)],
                gsem,
            )
            for j in range(NSTREAM)
        ]
        for h in handles:
            h.wait()
        pltpu.async_copy(
            rows_v, out.at[pl.ds(base + ci * CHUNK, CHUNK)], osem
        ).wait()
        return carry

    lax.fori_loop(0, NCHUNK, chunk_body, 0)


_posenc_call = functools.partial(
    pl.kernel,
    mesh=plsc.VectorSubcoreMesh(core_axis_name="c", subcore_axis_name="s"),
    out_type=jax.ShapeDtypeStruct((B_TOTAL, D), jnp.float32),
    scratch_types=[
        pltpu.VMEM((NSTREAM, IDXW), jnp.int32),
        pltpu.VMEM((CHUNK, D), jnp.float32),
        pltpu.SemaphoreType.DMA,
        pltpu.SemaphoreType.DMA,
        pltpu.SemaphoreType.DMA,
    ],
)(_posenc_body)


@jax.jit
def kernel(t, pos_enc):
    t2 = t.reshape(B_TOTAL // IDXW, IDXW)
    out = _posenc_call(t2, pos_enc)
    return out.reshape(t.shape + (D,))


# speedup vs baseline: 3.9592x; 3.9592x over previous
import functools

import jax
import jax.numpy as jnp
from jax import lax
from jax.experimental import pallas as pl
from jax.experimental.pallas import tpu as pltpu
from jax.experimental.pallas import tpu_sc as plsc

MAXP = 100000
D = 64
B_TOTAL = 4096 * 200
NC, NS, L = 2, 16, 16
NW = NC * NS
B_PER_W = B_TOTAL // NW
IDXW = 128
NSTREAM = 8
CHUNK = NSTREAM * IDXW
NCHUNK = B_PER_W // CHUNK
IROWS_PER_W = B_PER_W // IDXW


def _posenc_body(t2, table, out, idx_v, rows_v, isem, gsem, osem):
    wid = lax.axis_index("s") * NC + lax.axis_index("c")
    base = wid * B_PER_W
    irow0 = wid * IROWS_PER_W

    def chunk_body(ci, carry):
        pltpu.async_copy(
            t2.at[pl.ds(irow0 + ci * NSTREAM, NSTREAM)], idx_v, isem
        ).wait()
        for j in range(NSTREAM):
            for i in range(IDXW // L):
                v = idx_v[j, pl.ds(i * L, L)]
                idx_v[j, pl.ds(i * L, L)] = jnp.where(v == 0, MAXP - 1, v - 1)
        handles = [
            pltpu.async_copy(
                table.at[idx_v.at[j]],
                rows_v.at[pl.ds(j * IDXW, IDXW)],
                gsem,
            )
            for j in range(NSTREAM)
        ]
        for h in handles:
            h.wait()
        pltpu.async_copy(
            rows_v, out.at[pl.ds(base + ci * CHUNK, CHUNK)], osem
        ).wait()
        return carry

    lax.fori_loop(0, NCHUNK, chunk_body, 0)


_posenc_call = functools.partial(
    pl.kernel,
    mesh=plsc.VectorSubcoreMesh(core_axis_name="c", subcore_axis_name="s"),
    out_type=jax.ShapeDtypeStruct((B_TOTAL, D), jnp.float32),
    scratch_types=[
        pltpu.VMEM((NSTREAM, IDXW), jnp.int32),
        pltpu.VMEM((CHUNK, D), jnp.float32),
        pltpu.SemaphoreType.DMA,
        pltpu.SemaphoreType.DMA,
        pltpu.SemaphoreType.DMA,
    ],
    compiler_params=pltpu.CompilerParams(use_tc_tiling_on_sc=False),
)(_posenc_body)


@jax.jit
def kernel(t, pos_enc):
    t2 = t.reshape(B_TOTAL // IDXW, IDXW)
    out = _posenc_call(t2, pos_enc)
    return out.reshape(t.shape + (D,))


# speedup vs baseline: 4.0793x; 1.0303x over previous
import functools

import jax
import jax.numpy as jnp
from jax import lax
from jax.experimental import pallas as pl
from jax.experimental.pallas import tpu as pltpu
from jax.experimental.pallas import tpu_sc as plsc

MAXP = 100000
D = 64
B_TOTAL = 4096 * 200
NC, NS, L = 2, 16, 16
NW = NC * NS
B_PER_W = B_TOTAL // NW
IDXW = 128
NSTREAM = 5
CHUNK = NSTREAM * IDXW
NCHUNK = B_PER_W // CHUNK
NBODY = NCHUNK // 2
IROWS_PER_W = B_PER_W // IDXW


def _posenc_body(t2, table, out, idx_v, rows_v, isems, gsems, osems):
    wid = lax.axis_index("s") * NC + lax.axis_index("c")
    base = wid * B_PER_W
    irow0 = wid * IROWS_PER_W

    def idx_copy(ci, slot):
        return pltpu.make_async_copy(
            t2.at[pl.ds(irow0 + ci * NSTREAM, NSTREAM)],
            idx_v.at[slot],
            isems[slot],
        )

    def out_copy(ci, slot):
        return pltpu.make_async_copy(
            rows_v.at[slot],
            out.at[pl.ds(base + ci * CHUNK, CHUNK)],
            osems[slot],
        )

    def adjust(slot):
        for j in range(NSTREAM):
            for i in range(IDXW // L):
                v = idx_v[slot, j, pl.ds(i * L, L)]
                idx_v[slot, j, pl.ds(i * L, L)] = jnp.where(
                    v == 0, MAXP - 1, v - 1
                )

    def fire_gathers(slot):
        return [
            pltpu.async_copy(
                table.at[idx_v.at[slot, j]],
                rows_v.at[slot, pl.ds(j * IDXW, IDXW)],
                gsems[slot],
            )
            for j in range(NSTREAM)
        ]

    def half(k, ci, slot):
        idx_copy(ci, slot).wait()
        adjust(slot)
        @pl.when(k > 0)
        def _():
            out_copy(ci - 2, slot).wait()

        handles = fire_gathers(slot)
        if slot == 0:
            idx_copy(ci + 1, 1).start()
        else:
            @pl.when(k < NBODY - 1)
            def _():
                idx_copy(ci + 1, 0).start()

        for h in handles:
            h.wait()
        out_copy(ci, slot).start()

    def body(k, carry):
        half(k, 2 * k, 0)
        half(k, 2 * k + 1, 1)
        return carry

    idx_copy(0, 0).start()
    lax.fori_loop(0, NBODY, body, 0)
    out_copy(NCHUNK - 2, 0).wait()
    out_copy(NCHUNK - 1, 1).wait()


_posenc_call = functools.partial(
    pl.kernel,
    mesh=plsc.VectorSubcoreMesh(core_axis_name="c", subcore_axis_name="s"),
    out_type=jax.ShapeDtypeStruct((B_TOTAL, D), jnp.float32),
    scratch_types=[
        pltpu.VMEM((2, NSTREAM, IDXW), jnp.int32),
        pltpu.VMEM((2, CHUNK, D), jnp.float32),
        [pltpu.SemaphoreType.DMA] * 2,
        [pltpu.SemaphoreType.DMA] * 2,
        [pltpu.SemaphoreType.DMA] * 2,
    ],
    compiler_params=pltpu.CompilerParams(use_tc_tiling_on_sc=False),
)(_posenc_body)


@jax.jit
def kernel(t, pos_enc):
    t2 = t.reshape(B_TOTAL // IDXW, IDXW)
    out = _posenc_call(t2, pos_enc)
    return out.reshape(t.shape + (D,))


# speedup vs baseline: 5.0002x; 1.2257x over previous
import functools

import jax
import jax.numpy as jnp
from jax import lax
from jax.experimental import pallas as pl
from jax.experimental.pallas import tpu as pltpu
from jax.experimental.pallas import tpu_sc as plsc

MAXP = 100000
D = 64
DP = 128
B_TOTAL = 4096 * 200
NC, NS, L = 2, 16, 16
NW = NC * NS
B_PER_W = B_TOTAL // NW
IDXW = 128
NSTREAM = 4
HALF = NSTREAM * IDXW
CHUNK = 2 * HALF
NCHUNK = B_PER_W // CHUNK
IROWS_PER_W = B_PER_W // IDXW


def _posenc_body(t2, table, out, idx_v, rows_v, isem, gsem, osem):
    wid = lax.axis_index("s") * NC + lax.axis_index("c")
    base = wid * B_PER_W
    irow0 = wid * IROWS_PER_W

    def chunk_body(ci, carry):
        pltpu.async_copy(
            t2.at[pl.ds(irow0 + ci * 2 * NSTREAM, 2 * NSTREAM)], idx_v, isem
        ).wait()
        for j in range(2 * NSTREAM):
            for i in range(IDXW // L):
                v = idx_v[j, pl.ds(i * L, L)]
                idx_v[j, pl.ds(i * L, L)] = jnp.where(v == 0, MAXP - 1, v - 1)
        for h in range(2):
            handles = [
                pltpu.async_copy(
                    table.at[idx_v.at[h * NSTREAM + j]],
                    rows_v.at[pl.ds(j * IDXW, IDXW)],
                    gsem,
                )
                for j in range(NSTREAM)
            ]
            for hd in handles:
                hd.wait()
            pltpu.async_copy(
                rows_v, out.at[pl.ds(base + ci * CHUNK + h * HALF, HALF)], osem
            ).wait()
        return carry

    lax.fori_loop(0, NCHUNK, chunk_body, 0)


_posenc_call = functools.partial(
    pl.kernel,
    mesh=plsc.VectorSubcoreMesh(core_axis_name="c", subcore_axis_name="s"),
    out_type=jax.ShapeDtypeStruct((B_TOTAL, DP), jnp.float32),
    scratch_types=[
        pltpu.VMEM((2 * NSTREAM, IDXW), jnp.int32),
        pltpu.VMEM((HALF, DP), jnp.float32),
        pltpu.SemaphoreType.DMA,
        pltpu.SemaphoreType.DMA,
        pltpu.SemaphoreType.DMA,
    ],
)(_posenc_body)


@jax.jit
def kernel(t, pos_enc):
    t2 = t.reshape(B_TOTAL // IDXW, IDXW)
    pe128 = jnp.pad(pos_enc, ((0, 0), (0, DP - D)))
    out = _posenc_call(t2, pe128)
    return out[:, :D].reshape(t.shape + (D,))


# speedup vs baseline: 6.5393x; 1.3078x over previous
import functools

import jax
import jax.numpy as jnp
from jax import lax
from jax.experimental import pallas as pl
from jax.experimental.pallas import tpu as pltpu
from jax.experimental.pallas import tpu_sc as plsc

MAXP = 100000
D = 64
DP = 128
B_TOTAL = 4096 * 200
NC, NS, L = 2, 16, 16
NW = NC * NS
B_PER_W = B_TOTAL // NW
IDXW = 128
NSTREAM = 4
HALF = NSTREAM * IDXW
CHUNK = 2 * HALF
NCHUNK = B_PER_W // CHUNK
IROWS_PER_W = B_PER_W // IDXW


def _posenc_body(t2, table, out, idx_v, rows_v, isem, gsem, osem):
    wid = lax.axis_index("s") * NC + lax.axis_index("c")
    base = wid * B_PER_W
    irow0 = wid * IROWS_PER_W

    def chunk_body(ci, carry):
        pltpu.async_copy(
            t2.at[pl.ds(irow0 + ci * 2 * NSTREAM, 2 * NSTREAM)], idx_v, isem
        ).wait()
        for j in range(2 * NSTREAM):
            for i in range(IDXW // L):
                v = idx_v[j, pl.ds(i * L, L)]
                idx_v[j, pl.ds(i * L, L)] = jnp.where(v == 0, MAXP - 1, v - 1)
        for h in range(2):
            handles = [
                pltpu.async_copy(
                    table.at[idx_v.at[h * NSTREAM + j]],
                    rows_v.at[pl.ds(j * IDXW, IDXW)],
                    gsem,
                )
                for j in range(NSTREAM)
            ]
            for hd in handles:
                hd.wait()
            pltpu.async_copy(
                rows_v,
                out.at[pl.ds(base + ci * CHUNK + h * HALF, HALF), pl.ds(0, D)],
                osem,
            ).wait()
        return carry

    lax.fori_loop(0, NCHUNK, chunk_body, 0)


_posenc_call = functools.partial(
    pl.kernel,
    mesh=plsc.VectorSubcoreMesh(core_axis_name="c", subcore_axis_name="s"),
    out_type=jax.ShapeDtypeStruct((B_TOTAL, DP), jnp.float32),
    scratch_types=[
        pltpu.VMEM((2 * NSTREAM, IDXW), jnp.int32),
        pltpu.VMEM((HALF, D), jnp.float32),
        pltpu.SemaphoreType.DMA,
        pltpu.SemaphoreType.DMA,
        pltpu.SemaphoreType.DMA,
    ],
    compiler_params=pltpu.CompilerParams(use_tc_tiling_on_sc=False),
)(_posenc_body)


@jax.jit
def kernel(t, pos_enc):
    t2 = t.reshape(B_TOTAL // IDXW, IDXW)
    out = _posenc_call(t2, pos_enc)
    return out[:, :D].reshape(t.shape + (D,))


# speedup vs baseline: 7.1644x; 1.0956x over previous
import functools

import jax
import jax.numpy as jnp
from jax import lax
from jax.experimental import pallas as pl
from jax.experimental.pallas import tpu as pltpu
from jax.experimental.pallas import tpu_sc as plsc

MAXP = 100000
D = 64
DP = 128
B_TOTAL = 4096 * 200
NC, NS, L = 2, 16, 16
NW = NC * NS
B_PER_W = B_TOTAL // NW
IDXW = 128
NSTREAM = 4
HALF = NSTREAM * IDXW
CHUNK = 2 * HALF
NCHUNK = B_PER_W // CHUNK
IROWS_PER_W = B_PER_W // IDXW


def _posenc_body(t2, table, out, idx_v, rows_v, isem, gsems, osems):
    wid = lax.axis_index("s") * NC + lax.axis_index("c")
    base = wid * B_PER_W
    irow0 = wid * IROWS_PER_W

    def idx_copy(ci):
        return pltpu.make_async_copy(
            t2.at[pl.ds(irow0 + ci * 2 * NSTREAM, 2 * NSTREAM)], idx_v, isem
        )

    def out_copy(ci, h):
        return pltpu.make_async_copy(
            rows_v.at[h],
            out.at[pl.ds(base + ci * CHUNK + h * HALF, HALF), pl.ds(0, D)],
            osems[h],
        )

    def fire_gathers(h):
        return [
            pltpu.async_copy(
                table.at[idx_v.at[h * NSTREAM + j]],
                rows_v.at[h, pl.ds(j * IDXW, IDXW)],
                gsems[h],
            )
            for j in range(NSTREAM)
        ]

    def body(ci, carry):
        idx_copy(ci).wait()
        for j in range(2 * NSTREAM):
            for i in range(IDXW // L):
                v = idx_v[j, pl.ds(i * L, L)]
                idx_v[j, pl.ds(i * L, L)] = jnp.where(v == 0, MAXP - 1, v - 1)

        @pl.when(ci > 0)
        def _():
            out_copy(ci - 1, 0).wait()

        h0 = fire_gathers(0)

        @pl.when(ci > 0)
        def _():
            out_copy(ci - 1, 1).wait()

        h1 = fire_gathers(1)
        for hd in h0:
            hd.wait()
        out_copy(ci, 0).start()
        for hd in h1:
            hd.wait()

        @pl.when(ci < NCHUNK - 1)
        def _():
            idx_copy(ci + 1).start()

        out_copy(ci, 1).start()
        return carry

    idx_copy(0).start()
    lax.fori_loop(0, NCHUNK, body, 0)
    out_copy(NCHUNK - 1, 0).wait()
    out_copy(NCHUNK - 1, 1).wait()


_posenc_call = functools.partial(
    pl.kernel,
    mesh=plsc.VectorSubcoreMesh(core_axis_name="c", subcore_axis_name="s"),
    out_type=jax.ShapeDtypeStruct((B_TOTAL, DP), jnp.float32),
    scratch_types=[
        pltpu.VMEM((2 * NSTREAM, IDXW), jnp.int32),
        pltpu.VMEM((2, HALF, D), jnp.float32),
        pltpu.SemaphoreType.DMA,
        [pltpu.SemaphoreType.DMA] * 2,
        [pltpu.SemaphoreType.DMA] * 2,
    ],
    compiler_params=pltpu.CompilerParams(use_tc_tiling_on_sc=False),
)(_posenc_body)


@jax.jit
def kernel(t, pos_enc):
    t2 = t.reshape(B_TOTAL // IDXW, IDXW)
    out = _posenc_call(t2, pos_enc)
    return out[:, :D].reshape(t.shape + (D,))
